# baseline (speedup 1.0000x reference)
import functools

import jax
import jax.numpy as jnp
from jax import lax
from jax.experimental import pallas as pl
from jax.experimental.pallas import tpu as pltpu
from jax.experimental.pallas import tpu_sc as plsc

N = 100000
E = 1600000
IN_DIM = 5
HIDDEN = 32
OUT_DIM = 2

HALF = 16
CW = 128
NPAD = 100096
ROWS_PER_TILE = NPAD // 16

Q_E = 200
Q_F = 400
P = 20
NPAIRPAD = 6420
EPAD = NPAIRPAD * 2 * CW

_mesh = plsc.VectorSubcoreMesh(core_axis_name="c", subcore_axis_name="s")


def _make_agg(feature_split):
  Q = Q_F if feature_split else Q_E
  PH = Q // P
  NI = P // 2

  @functools.partial(
      pl.kernel,
      out_type=jax.ShapeDtypeStruct((2, NPAD, HALF), jnp.float32),
      mesh=_mesh,
      compiler_params=pltpu.CompilerParams(use_tc_tiling_on_sc=False),
      scratch_types=[
          pltpu.VMEM((P, 2, CW), jnp.int32),
          pltpu.VMEM((P, 2, CW), jnp.int32),
          pltpu.VMEM((P, 2, CW), jnp.int32),
          pltpu.VMEM((P, 2, CW), jnp.int32),
          pltpu.VMEM((2, CW, HALF), jnp.float32),
          pltpu.VMEM((2, CW, HALF), jnp.float32),
          pltpu.VMEM_SHARED((NPAD, HALF), jnp.float32),
          pltpu.SemaphoreType.DMA,
          pltpu.SemaphoreType.DMA,
          pltpu.SemaphoreType.DMA,
          pltpu.SemaphoreType.DMA,
          pltpu.SemaphoreType.DMA,
          pltpu.SemaphoreType.DMA,
      ],
  )
  def agg(table_hbm, src_hbm, dst_hbm, zero_hbm, out_hbm,
          isrc0, idst0, isrc1, idst1, rowsA, rowsB, acc,
          isem0, isem1, gsemA, gsemB, ssemA, ssemB):
    cid = lax.axis_index("c")
    sid = lax.axis_index("s")
    zlo = sid * ROWS_PER_TILE
    pltpu.sync_copy(zero_hbm.at[pl.ds(zlo, ROWS_PER_TILE)],
                    acc.at[pl.ds(zlo, ROWS_PER_TILE)])
    plsc.subcore_barrier()

    if feature_split:
      row0 = sid * Q
      splane = src_hbm.at[cid]
    else:
      row0 = (cid * 16 + sid) * Q
      splane = src_hbm.at[0]

    def fetch_idx(pbase, isrc, idst, isem):
      pltpu.async_copy(splane.at[pl.ds(pbase, P)], isrc, isem)
      pltpu.async_copy(dst_hbm.at[pl.ds(pbase, P)], idst, isem)

    def drain_idx(isrc, idst, isem):
      pltpu.make_async_copy(splane.at[pl.ds(row0, P)], isrc, isem).wait()
      pltpu.make_async_copy(dst_hbm.at[pl.ds(row0, P)], idst, isem).wait()

    def run_phase(isrc, idst):
      @pl.loop(0, NI)
      def _(sub):
        jj = 2 * sub
        gA = [pltpu.async_copy(table_hbm.at[isrc.at[jj, k]],
                               rowsA.at[k], gsemA) for k in range(2)]
        gB = [pltpu.async_copy(table_hbm.at[isrc.at[jj + 1, k]],
                               rowsB.at[k], gsemB) for k in range(2)]
        for g in gA:
          g.wait()
        sA = [pltpu.async_copy(rowsA.at[k], acc.at[idst.at[jj, k]],
                               ssemA, add=True) for k in range(2)]
        for g in gB:
          g.wait()
        sB = [pltpu.async_copy(rowsB.at[k], acc.at[idst.at[jj + 1, k]],
                               ssemB, add=True) for k in range(2)]
        for c in sA:
          c.wait()
        for c in sB:
          c.wait()

    fetch_idx(row0, isrc0, idst0, isem0)

    @pl.loop(0, PH, step=2)
    def _(ph):
      pbase = row0 + ph * P
      drain_idx(isrc0, idst0, isem0)
      fetch_idx(pbase + P, isrc1, idst1, isem1)
      run_phase(isrc0, idst0)
      drain_idx(isrc1, idst1, isem1)
      fetch_idx(pbase + 2 * P, isrc0, idst0, isem0)
      run_phase(isrc1, idst1)

    drain_idx(isrc0, idst0, isem0)

    plsc.subcore_barrier()
    pltpu.sync_copy(acc.at[pl.ds(zlo, ROWS_PER_TILE)],
                    out_hbm.at[cid].at[pl.ds(zlo, ROWS_PER_TILE)])

  return agg


@functools.partial(
    pl.kernel,
    out_type=jax.ShapeDtypeStruct((2, NPAD, HALF), jnp.float32),
    mesh=_mesh,
    compiler_params=pltpu.CompilerParams(use_tc_tiling_on_sc=False),
    scratch_types=[
        pltpu.VMEM((P, 2, CW), jnp.int32),
        pltpu.VMEM((P, 2, CW), jnp.int32),
        pltpu.VMEM((CW, HALF), jnp.float32),
        pltpu.VMEM_SHARED((NPAD, HALF), jnp.float32),
        pltpu.SemaphoreType.DMA,
        pltpu.SemaphoreType.DMA,
        pltpu.SemaphoreType.DMA,
    ],
)
def _degree_kernel(dst_hbm, ones_hbm, zero_hbm, out_hbm,
                   idst0, idst1, ones_v, acc, isem0, isem1, ssem):
  cid = lax.axis_index("c")
  sid = lax.axis_index("s")
  zlo = sid * ROWS_PER_TILE
  pltpu.sync_copy(zero_hbm.at[pl.ds(zlo, ROWS_PER_TILE)],
                  acc.at[pl.ds(zlo, ROWS_PER_TILE)])
  pltpu.sync_copy(ones_hbm, ones_v)
  plsc.subcore_barrier()

  row0 = (cid * 16 + sid) * Q_E
  PH = Q_E // P
  NI = P // 4

  def fetch_idx(pbase, idst, isem):
    pltpu.async_copy(dst_hbm.at[pl.ds(pbase, P)], idst, isem)

  def drain_idx(idst, isem):
    pltpu.make_async_copy(dst_hbm.at[pl.ds(row0, P)], idst, isem).wait()

  def run_phase(idst):
    @pl.loop(0, NI)
    def _(sub):
      jj = 4 * sub
      sc = [pltpu.async_copy(ones_v, acc.at[idst.at[jj + j, k]], ssem,
                             add=True)
            for j in range(4) for k in range(2)]
      for c in sc:
        c.wait()

  fetch_idx(row0, idst0, isem0)

  @pl.loop(0, PH, step=2)
  def _(ph):
    pbase = row0 + ph * P
    drain_idx(idst0, isem0)
    fetch_idx(pbase + P, idst1, isem1)
    run_phase(idst0)
    drain_idx(idst1, isem1)
    fetch_idx(pbase + 2 * P, idst0, isem0)
    run_phase(idst1)

  drain_idx(idst0, isem0)

  plsc.subcore_barrier()
  pltpu.sync_copy(acc.at[pl.ds(zlo, ROWS_PER_TILE)],
                  out_hbm.at[cid].at[pl.ds(zlo, ROWS_PER_TILE)])


_agg_edge_split = _make_agg(feature_split=False)
_agg_feat_split = _make_agg(feature_split=True)


BLK = 2000
GRID = N // BLK


def _tc_prep(degp, x):
  def body(degp_ref, x_ref, dinv_ref, xp_ref):
    deg = degp_ref[0, :, 0] + degp_ref[1, :, 0] + 1.0
    dinv = lax.rsqrt(jnp.maximum(deg, 1e-12))
    dinv_ref[...] = dinv[:, None]
    xs = x_ref[...] * dinv[:, None]
    xp_ref[...] = jnp.concatenate(
        [xs, jnp.zeros((BLK, HALF - IN_DIM), jnp.float32)], axis=1)

  return pl.pallas_call(
      body,
      grid=(GRID,),
      in_specs=[
          pl.BlockSpec((2, BLK, HALF), lambda i: (0, i, 0)),
          pl.BlockSpec((BLK, IN_DIM), lambda i: (i, 0)),
      ],
      out_specs=[
          pl.BlockSpec((BLK, 1), lambda i: (i, 0)),
          pl.BlockSpec((BLK, HALF), lambda i: (i, 0)),
      ],
      out_shape=[
          jax.ShapeDtypeStruct((N, 1), jnp.float32),
          jax.ShapeDtypeStruct((NPAD, HALF), jnp.float32),
      ],
  )(degp, x)


def _tc_layer1(aggp, x, dinv, W1p, b1):
  def body(agg_ref, x_ref, dinv_ref, w_ref, b_ref, out_ref):
    dv = dinv_ref[...]
    a = agg_ref[0] + agg_ref[1]
    xpad = jnp.concatenate(
        [x_ref[...], jnp.zeros((BLK, HALF - IN_DIM), jnp.float32)], axis=1)
    s = dv * a + (dv * dv) * xpad
    h = jnp.dot(s, w_ref[...], preferred_element_type=jnp.float32)
    h = jnp.maximum(h + b_ref[...][None, :], 0.0)
    hp = dv * h
    out_ref[0] = hp[:, :HALF]
    out_ref[1] = hp[:, HALF:]

  return pl.pallas_call(
      body,
      grid=(GRID,),
      in_specs=[
          pl.BlockSpec((2, BLK, HALF), lambda i: (0, i, 0)),
          pl.BlockSpec((BLK, IN_DIM), lambda i: (i, 0)),
          pl.BlockSpec((BLK, 1), lambda i: (i, 0)),
          pl.BlockSpec((HALF, HIDDEN), lambda i: (0, 0)),
          pl.BlockSpec((HIDDEN,), lambda i: (0,)),
      ],
      out_specs=pl.BlockSpec((2, BLK, HALF), lambda i: (0, i, 0)),
      out_shape=jax.ShapeDtypeStruct((2, NPAD, HALF), jnp.float32),
  )(aggp, x, dinv, W1p, b1)


def _tc_mid(aggp, hp_prev, dinv, W, b):
  def body(agg_ref, hp_ref, dinv_ref, w_ref, b_ref, out_ref):
    dv = dinv_ref[...]
    a = jnp.concatenate([agg_ref[0], agg_ref[1]], axis=1)
    hpc = jnp.concatenate([hp_ref[0], hp_ref[1]], axis=1)
    s = dv * (a + hpc)
    h = jnp.dot(s, w_ref[...], preferred_element_type=jnp.float32)
    h = jnp.maximum(h + b_ref[...][None, :], 0.0)
    hp = dv * h
    out_ref[0] = hp[:, :HALF]
    out_ref[1] = hp[:, HALF:]

  return pl.pallas_call(
      body,
      grid=(GRID,),
      in_specs=[
          pl.BlockSpec((2, BLK, HALF), lambda i: (0, i, 0)),
          pl.BlockSpec((2, BLK, HALF), lambda i: (0, i, 0)),
          pl.BlockSpec((BLK, 1), lambda i: (i, 0)),
          pl.BlockSpec((HIDDEN, HIDDEN), lambda i: (0, 0)),
          pl.BlockSpec((HIDDEN,), lambda i: (0,)),
      ],
      out_specs=pl.BlockSpec((2, BLK, HALF), lambda i: (0, i, 0)),
      out_shape=jax.ShapeDtypeStruct((2, NPAD, HALF), jnp.float32),
  )(aggp, hp_prev, dinv, W, b)


def _tc_final(aggp, hp_prev, dinv, W3, b3):
  def body(agg_ref, hp_ref, dinv_ref, w_ref, b_ref, out_ref):
    dv = dinv_ref[...]
    a = jnp.concatenate([agg_ref[0], agg_ref[1]], axis=1)
    hpc = jnp.concatenate([hp_ref[0], hp_ref[1]], axis=1)
    s = dv * (a + hpc)
    o = jnp.dot(s, w_ref[...], preferred_element_type=jnp.float32)
    out_ref[...] = o + b_ref[...][None, :]

  return pl.pallas_call(
      body,
      grid=(GRID,),
      in_specs=[
          pl.BlockSpec((2, BLK, HALF), lambda i: (0, i, 0)),
          pl.BlockSpec((2, BLK, HALF), lambda i: (0, i, 0)),
          pl.BlockSpec((BLK, 1), lambda i: (i, 0)),
          pl.BlockSpec((HIDDEN, OUT_DIM), lambda i: (0, 0)),
          pl.BlockSpec((OUT_DIM,), lambda i: (0,)),
      ],
      out_specs=pl.BlockSpec((BLK, OUT_DIM), lambda i: (i, 0)),
      out_shape=jax.ShapeDtypeStruct((N, OUT_DIM), jnp.float32),
  )(aggp, hp_prev, dinv, W3, b3)


def kernel(x, edge_index, W1, b1, W2, b2, W3, b3):
  src = edge_index[0]
  dst = edge_index[1]
  pad = jnp.full((EPAD - E,), N, dtype=jnp.int32)
  srcp = jnp.concatenate([src, pad]).reshape(NPAIRPAD, 2, CW)
  src2 = jnp.stack([srcp, srcp + NPAD])
  dst3 = jnp.concatenate([dst, pad]).reshape(NPAIRPAD, 2, CW)

  zeros = jnp.zeros((NPAD, HALF), jnp.float32)
  ones128 = jnp.ones((CW, HALF), jnp.float32)
  W1p = jnp.concatenate(
      [W1, jnp.zeros((HALF - IN_DIM, HIDDEN), jnp.float32)], axis=0)

  degp = _degree_kernel(dst3, ones128, zeros)
  dinv, xp = _tc_prep(degp, x)

  agg1 = _agg_edge_split(xp, src2, dst3, zeros)
  hp1 = _tc_layer1(agg1, x, dinv, W1p, b1)

  agg2 = _agg_feat_split(hp1.reshape(2 * NPAD, HALF), src2, dst3, zeros)
  hp2 = _tc_mid(agg2, hp1, dinv, W2, b2)

  agg3 = _agg_feat_split(hp2.reshape(2 * NPAD, HALF), src2, dst3, zeros)
  return _tc_final(agg3, hp2, dinv, W3, b3)
